# baseline (speedup 1.0000x reference)
import functools

import jax
import jax.numpy as jnp
from jax import lax
from jax.experimental import pallas as pl
from jax.experimental.pallas import tpu as pltpu
from jax.experimental.pallas import tpu_sc as plsc

N = 1048576
LODS = ((32, 0), (64, 32768), (128, 294912), (256, 2392064))

NW = 32
PW = N // NW
C = 512
NCH = PW // C
VPC = C // 16


def _sc_body(x_hbm, tab_in, out_hbm,
             xbuf, i0, i1, i2, i3, f0, f1, f2, f3,
             r0, r1, r2, r3, obuf, s0, s1, s2, s3):
    wid = lax.axis_index("s") * 2 + lax.axis_index("c")
    idxb = (i0, i1, i2, i3)
    fracb = (f0, f1, f2, f3)
    rowsb = (r0, r1, r2, r3)
    sems = (s0, s1, s2, s3)
    lanes = jnp.arange(16, dtype=jnp.int32)

    def chunk_body(k, carry):
        base = wid * PW + k * C
        pltpu.sync_copy(x_hbm.at[pl.ds(base * 3, C * 3)], xbuf)

        def gen(i, c2):
            l0 = i * 16
            p = (lanes + l0) * 3
            x = plsc.load_gather(xbuf, [p])
            y = plsc.load_gather(xbuf, [p + 1])
            z = plsc.load_gather(xbuf, [p + 2])
            for (R, off), ib, fb in zip(LODS, idxb, fracb):
                px = (x * 0.5 + 0.5) * (R - 1)
                py = (y * 0.5 + 0.5) * (R - 1)
                pz = (z * 0.5 + 0.5) * (R - 1)
                x0 = jnp.minimum(px.astype(jnp.int32), R - 2)
                y0 = jnp.minimum(py.astype(jnp.int32), R - 2)
                z0 = jnp.minimum(pz.astype(jnp.int32), R - 2)
                fb[pl.ds(l0, 16)] = px - x0.astype(jnp.float32)
                fb[pl.ds(C + l0, 16)] = py - y0.astype(jnp.float32)
                fb[pl.ds(2 * C + l0, 16)] = pz - z0.astype(jnp.float32)
                b = ((x0 * R + y0) * R + z0 + off) * 2
                for ci, d in enumerate(
                        (0, 1, R, R + 1, R * R, R * R + 1,
                         R * R + R, R * R + R + 1)):
                    ib[pl.ds((2 * ci) * C + l0, 16)] = b + 2 * d
                    ib[pl.ds((2 * ci + 1) * C + l0, 16)] = b + (2 * d + 1)
            return c2

        lax.fori_loop(0, VPC, gen, 0)

        copies = [pltpu.async_copy(tab_in.at[idxb[l]], rowsb[l], sems[l])
                  for l in range(4)]

        for l in range(4):
            copies[l].wait()
            fb, rb = fracb[l], rowsb[l]

            def comb(i, c2, l=l, fb=fb, rb=rb):
                l0 = i * 16
                lane = lanes + l0
                fx = fb[pl.ds(l0, 16)]
                fy = fb[pl.ds(C + l0, 16)]
                fz = fb[pl.ds(2 * C + l0, 16)]
                v = []
                for c in range(8):
                    v.append((plsc.load_gather(rb, [lane + (2 * c) * C]),
                              plsc.load_gather(rb, [lane + (2 * c + 1) * C])))
                for f in range(2):
                    c00 = v[0][f] + fz * (v[1][f] - v[0][f])
                    c01 = v[2][f] + fz * (v[3][f] - v[2][f])
                    c10 = v[4][f] + fz * (v[5][f] - v[4][f])
                    c11 = v[6][f] + fz * (v[7][f] - v[6][f])
                    c0 = c00 + fy * (c01 - c00)
                    c1 = c10 + fy * (c11 - c10)
                    val = c0 + fx * (c1 - c0)
                    plsc.store_scatter(
                        obuf, [lane, jnp.full((16,), 2 * l + f, jnp.int32)],
                        val)
                return c2

            lax.fori_loop(0, VPC, comb, 0)

        pltpu.sync_copy(obuf, out_hbm.at[pl.ds(base, C)])
        return carry

    lax.fori_loop(0, NCH, chunk_body, 0)


@functools.cache
def _make_sc_forward():
    return functools.partial(
        pl.kernel,
        mesh=plsc.VectorSubcoreMesh(core_axis_name="c", subcore_axis_name="s"),
        out_type=jax.ShapeDtypeStruct((N, 8), jnp.float32),
        compiler_params=pltpu.CompilerParams(
            needs_layout_passes=False, use_tc_tiling_on_sc=False),
        scratch_types=[
            pltpu.VMEM((C * 3,), jnp.float32),
            pltpu.VMEM((16 * C,), jnp.int32),
            pltpu.VMEM((16 * C,), jnp.int32),
            pltpu.VMEM((16 * C,), jnp.int32),
            pltpu.VMEM((16 * C,), jnp.int32),
            pltpu.VMEM((3 * C,), jnp.float32),
            pltpu.VMEM((3 * C,), jnp.float32),
            pltpu.VMEM((3 * C,), jnp.float32),
            pltpu.VMEM((3 * C,), jnp.float32),
            pltpu.VMEM((16 * C,), jnp.float32),
            pltpu.VMEM((16 * C,), jnp.float32),
            pltpu.VMEM((16 * C,), jnp.float32),
            pltpu.VMEM((16 * C,), jnp.float32),
            pltpu.VMEM((C, 8), jnp.float32),
            pltpu.SemaphoreType.DMA,
            pltpu.SemaphoreType.DMA,
            pltpu.SemaphoreType.DMA,
            pltpu.SemaphoreType.DMA,
        ],
    )(_sc_body)


def kernel(input, flattened_params):
    return _make_sc_forward()(input.reshape(N * 3), flattened_params)


# speedup vs baseline: 1.0920x; 1.0920x over previous
import functools

import jax
import jax.numpy as jnp
from jax import lax
from jax.experimental import pallas as pl
from jax.experimental.pallas import tpu as pltpu
from jax.experimental.pallas import tpu_sc as plsc

N = 1048576
L0_SIZE = 65536
T12_ROWS = 589824
T12_OFF = 65536
L3_ELEM_OFF = 4784128

LOD0 = (32, 0)
LOD12 = ((64, 32768), (128, 294912))
LOD3 = (256, 2392064)

NW = 32
PW = N // NW
C = 256
NCH = PW // C
VPC = C // 16


def _corner_offsets(R):
    return (0, 1, R, R + 1, R * R, R * R + 1, R * R + R, R * R + R + 1)


def _sc_body(x_hbm, t12_hbm, raw_hbm, out_hbm,
             l0tab, xbuf, pb0, fr0, i1, fr1, i2, fr2, i3, fr3,
             r1, r2, r3, obuf, s1, s2, s3):
    wid = lax.axis_index("s") * 2 + lax.axis_index("c")
    lanes = jnp.arange(16, dtype=jnp.int32)

    pltpu.sync_copy(raw_hbm.at[pl.ds(0, L0_SIZE)], l0tab)

    def chunk_body(k, carry):
        base = wid * PW + k * C
        pltpu.sync_copy(x_hbm.at[pl.ds(base * 3, C * 3)], xbuf)

        def gen(i, c2):
            l0 = i * 16
            p = (lanes + l0) * 3
            x = plsc.load_gather(xbuf, [p])
            y = plsc.load_gather(xbuf, [p + 1])
            z = plsc.load_gather(xbuf, [p + 2])

            def prep(R, off):
                px = (x * 0.5 + 0.5) * (R - 1)
                py = (y * 0.5 + 0.5) * (R - 1)
                pz = (z * 0.5 + 0.5) * (R - 1)
                x0 = jnp.minimum(px.astype(jnp.int32), R - 2)
                y0 = jnp.minimum(py.astype(jnp.int32), R - 2)
                z0 = jnp.minimum(pz.astype(jnp.int32), R - 2)
                fx = px - x0.astype(jnp.float32)
                fy = py - y0.astype(jnp.float32)
                fz = pz - z0.astype(jnp.float32)
                b = (x0 * R + y0) * R + z0 + off
                return b, z0, fx, fy, fz

            R, off = LOD0
            b, _, fx, fy, fz = prep(R, off)
            fr0[pl.ds(l0, 16)] = fx
            fr0[pl.ds(C + l0, 16)] = fy
            fr0[pl.ds(2 * C + l0, 16)] = fz
            pb0[pl.ds(l0, 16)] = b

            for (R, off), ib, fb in zip(LOD12, (i1, i2), (fr1, fr2)):
                b, z0, fx, fy, fz = prep(R, off)
                fb[pl.ds(l0, 16)] = fx
                fb[pl.ds(C + l0, 16)] = fy
                fb[pl.ds(2 * C + l0, 16)] = fz
                fb[pl.ds(3 * C + l0, 16)] = (
                    jnp.bitwise_and(z0, 3).astype(jnp.float32))
                bl = b - T12_OFF // 2
                for ci, d in enumerate(_corner_offsets(R)):
                    ib[pl.ds(ci * C + l0, 16)] = jnp.right_shift(bl + d, 2)

            R, off = LOD3
            b, _, fx, fy, fz = prep(R, off)
            fr3[pl.ds(l0, 16)] = fx
            fr3[pl.ds(C + l0, 16)] = fy
            fr3[pl.ds(2 * C + l0, 16)] = fz
            e = b * 2
            for ci, d in enumerate(_corner_offsets(R)):
                i3[pl.ds((2 * ci) * C + l0, 16)] = e + 2 * d
                i3[pl.ds((2 * ci + 1) * C + l0, 16)] = e + (2 * d + 1)
            return c2

        lax.fori_loop(0, VPC, gen, 0)

        c1 = pltpu.async_copy(t12_hbm.at[i1], r1, s1)
        c2_ = pltpu.async_copy(t12_hbm.at[i2], r2, s2)
        c3 = pltpu.async_copy(raw_hbm.at[i3], r3, s3)

        def lerp(v, fx, fy, fz, f):
            c00 = v[0][f] + fz * (v[1][f] - v[0][f])
            c01 = v[2][f] + fz * (v[3][f] - v[2][f])
            c10 = v[4][f] + fz * (v[5][f] - v[4][f])
            c11 = v[6][f] + fz * (v[7][f] - v[6][f])
            c0 = c00 + fy * (c01 - c00)
            c1_ = c10 + fy * (c11 - c10)
            return c0 + fx * (c1_ - c0)

        def comb0(i, c2):
            l0 = i * 16
            lane = lanes + l0
            fx = fr0[pl.ds(l0, 16)]
            fy = fr0[pl.ds(C + l0, 16)]
            fz = fr0[pl.ds(2 * C + l0, 16)]
            pb = pb0[pl.ds(l0, 16)]
            v = []
            for d in _corner_offsets(LOD0[0]):
                e = (pb + d) * 2
                v.append((plsc.load_gather(l0tab, [e]),
                          plsc.load_gather(l0tab, [e + 1])))
            for f in range(2):
                plsc.store_scatter(
                    obuf, [lane, jnp.full((16,), f, jnp.int32)],
                    lerp(v, fx, fy, fz, f))
            return c2

        lax.fori_loop(0, VPC, comb0, 0)

        for li, (cp, fb, rb) in enumerate(((c1, fr1, r1), (c2_, fr2, r2))):
            cp.wait()

            def comb12(i, c2, li=li, fb=fb, rb=rb):
                l0 = i * 16
                lane = lanes + l0
                fx = fb[pl.ds(l0, 16)]
                fy = fb[pl.ds(C + l0, 16)]
                fz = fb[pl.ds(2 * C + l0, 16)]
                zlow = fb[pl.ds(3 * C + l0, 16)].astype(jnp.int32)
                czero = zlow * 2
                cone = jnp.bitwise_and(zlow + 1, 3) * 2
                v = []
                for c in range(8):
                    r = lane + c * C
                    col = cone if (c & 1) else czero
                    v.append((plsc.load_gather(rb, [r, col]),
                              plsc.load_gather(rb, [r, col + 1])))
                for f in range(2):
                    plsc.store_scatter(
                        obuf,
                        [lane, jnp.full((16,), 2 * (li + 1) + f, jnp.int32)],
                        lerp(v, fx, fy, fz, f))
                return c2

            lax.fori_loop(0, VPC, comb12, 0)

        c3.wait()

        def comb3(i, c2):
            l0 = i * 16
            lane = lanes + l0
            fx = fr3[pl.ds(l0, 16)]
            fy = fr3[pl.ds(C + l0, 16)]
            fz = fr3[pl.ds(2 * C + l0, 16)]
            v = []
            for c in range(8):
                v.append((plsc.load_gather(r3, [lane + (2 * c) * C]),
                          plsc.load_gather(r3, [lane + (2 * c + 1) * C])))
            for f in range(2):
                plsc.store_scatter(
                    obuf, [lane, jnp.full((16,), 6 + f, jnp.int32)],
                    lerp(v, fx, fy, fz, f))
            return c2

        lax.fori_loop(0, VPC, comb3, 0)

        pltpu.sync_copy(obuf, out_hbm.at[pl.ds(base, C)])
        return carry

    lax.fori_loop(0, NCH, chunk_body, 0)


@functools.cache
def _make_sc_forward():
    return functools.partial(
        pl.kernel,
        mesh=plsc.VectorSubcoreMesh(core_axis_name="c", subcore_axis_name="s"),
        out_type=jax.ShapeDtypeStruct((N, 8), jnp.float32),
        compiler_params=pltpu.CompilerParams(
            needs_layout_passes=False, use_tc_tiling_on_sc=False),
        scratch_types=[
            pltpu.VMEM((L0_SIZE,), jnp.float32),
            pltpu.VMEM((C * 3,), jnp.float32),
            pltpu.VMEM((C,), jnp.int32),
            pltpu.VMEM((3 * C,), jnp.float32),
            pltpu.VMEM((8 * C,), jnp.int32),
            pltpu.VMEM((4 * C,), jnp.float32),
            pltpu.VMEM((8 * C,), jnp.int32),
            pltpu.VMEM((4 * C,), jnp.float32),
            pltpu.VMEM((16 * C,), jnp.int32),
            pltpu.VMEM((3 * C,), jnp.float32),
            pltpu.VMEM((8 * C, 8), jnp.float32),
            pltpu.VMEM((8 * C, 8), jnp.float32),
            pltpu.VMEM((16 * C,), jnp.float32),
            pltpu.VMEM((C, 8), jnp.float32),
            pltpu.SemaphoreType.DMA,
            pltpu.SemaphoreType.DMA,
            pltpu.SemaphoreType.DMA,
        ],
    )(_sc_body)


def kernel(input, flattened_params):
    t12 = flattened_params[T12_OFF:L3_ELEM_OFF].reshape(T12_ROWS, 8)
    return _make_sc_forward()(
        input.reshape(N * 3), t12, flattened_params)


# speedup vs baseline: 1.4629x; 1.3397x over previous
import functools

import jax
import jax.numpy as jnp
from jax import lax
from jax.experimental import pallas as pl
from jax.experimental.pallas import tpu as pltpu
from jax.experimental.pallas import tpu_sc as plsc

N = 1048576
N_V8 = 4792320
L0_ROWS = 8192
LOD0 = (32, 0)
LOD123 = ((64, 32768), (128, 294912), (256, 2392064))

NW = 32
PW = N // NW
C = 256
NCH = PW // C
VPC = C // 16


def _corner_offsets(R):
    return (0, 1, R, R + 1, R * R, R * R + 1, R * R + R, R * R + R + 1)


def _sc_body(x_hbm, tab_hbm, out_hbm,
             l0tab, xbuf, pb0, fr0, i1, fr1, i2, fr2, i3, fr3,
             r1, r2, r3, obuf, s1, s2, s3):
    wid = lax.axis_index("s") * 2 + lax.axis_index("c")
    lanes = jnp.arange(16, dtype=jnp.int32)

    pltpu.sync_copy(tab_hbm.at[pl.ds(0, L0_ROWS)], l0tab)

    def chunk_body(k, carry):
        base = wid * PW + k * C
        pltpu.sync_copy(x_hbm.at[pl.ds(base * 3, C * 3)], xbuf)

        def gen(i, c2):
            l0 = i * 16
            p = (lanes + l0) * 3
            x = plsc.load_gather(xbuf, [p])
            y = plsc.load_gather(xbuf, [p + 1])
            z = plsc.load_gather(xbuf, [p + 2])

            def prep(R, off):
                px = (x * 0.5 + 0.5) * (R - 1)
                py = (y * 0.5 + 0.5) * (R - 1)
                pz = (z * 0.5 + 0.5) * (R - 1)
                x0 = jnp.minimum(px.astype(jnp.int32), R - 2)
                y0 = jnp.minimum(py.astype(jnp.int32), R - 2)
                z0 = jnp.minimum(pz.astype(jnp.int32), R - 2)
                fx = px - x0.astype(jnp.float32)
                fy = py - y0.astype(jnp.float32)
                fz = pz - z0.astype(jnp.float32)
                b = (x0 * R + y0) * R + z0 + off
                return b, z0, fx, fy, fz

            R, off = LOD0
            b, _, fx, fy, fz = prep(R, off)
            fr0[pl.ds(l0, 16)] = fx
            fr0[pl.ds(C + l0, 16)] = fy
            fr0[pl.ds(2 * C + l0, 16)] = fz
            pb0[pl.ds(l0, 16)] = b

            for (R, off), ib, fb in zip(LOD123, (i1, i2, i3), (fr1, fr2, fr3)):
                b, z0, fx, fy, fz = prep(R, off)
                fb[pl.ds(l0, 16)] = fx
                fb[pl.ds(C + l0, 16)] = fy
                fb[pl.ds(2 * C + l0, 16)] = fz
                fb[pl.ds(3 * C + l0, 16)] = (
                    jnp.bitwise_and(z0, 3).astype(jnp.float32))
                for ci, d in enumerate(_corner_offsets(R)):
                    ib[pl.ds(ci * C + l0, 16)] = jnp.right_shift(b + d, 2)
            return c2

        lax.fori_loop(0, VPC, gen, 0)

        copies = [pltpu.async_copy(tab_hbm.at[ib], rb, sm)
                  for ib, rb, sm in ((i1, r1, s1), (i2, r2, s2), (i3, r3, s3))]

        def lerp(v, fx, fy, fz, f):
            c00 = v[0][f] + fz * (v[1][f] - v[0][f])
            c01 = v[2][f] + fz * (v[3][f] - v[2][f])
            c10 = v[4][f] + fz * (v[5][f] - v[4][f])
            c11 = v[6][f] + fz * (v[7][f] - v[6][f])
            c0 = c00 + fy * (c01 - c00)
            c1_ = c10 + fy * (c11 - c10)
            return c0 + fx * (c1_ - c0)

        def comb0(i, c2):
            l0 = i * 16
            lane = lanes + l0
            fx = fr0[pl.ds(l0, 16)]
            fy = fr0[pl.ds(C + l0, 16)]
            fz = fr0[pl.ds(2 * C + l0, 16)]
            pb = pb0[pl.ds(l0, 16)]
            v = []
            for d in _corner_offsets(LOD0[0]):
                e = pb + d
                row = jnp.right_shift(e, 2)
                col = jnp.bitwise_and(e, 3) * 2
                v.append((plsc.load_gather(l0tab, [row, col]),
                          plsc.load_gather(l0tab, [row, col + 1])))
            for f in range(2):
                plsc.store_scatter(
                    obuf, [lane, jnp.full((16,), f, jnp.int32)],
                    lerp(v, fx, fy, fz, f))
            return c2

        lax.fori_loop(0, VPC, comb0, 0)

        for li, (cp, fb, rb) in enumerate(
                zip(copies, (fr1, fr2, fr3), (r1, r2, r3))):
            cp.wait()

            def comb(i, c2, li=li, fb=fb, rb=rb):
                l0 = i * 16
                lane = lanes + l0
                fx = fb[pl.ds(l0, 16)]
                fy = fb[pl.ds(C + l0, 16)]
                fz = fb[pl.ds(2 * C + l0, 16)]
                zlow = fb[pl.ds(3 * C + l0, 16)].astype(jnp.int32)
                czero = zlow * 2
                cone = jnp.bitwise_and(zlow + 1, 3) * 2
                v = []
                for c in range(8):
                    r = lane + c * C
                    col = cone if (c & 1) else czero
                    v.append((plsc.load_gather(rb, [r, col]),
                              plsc.load_gather(rb, [r, col + 1])))
                for f in range(2):
                    plsc.store_scatter(
                        obuf,
                        [lane, jnp.full((16,), 2 * (li + 1) + f, jnp.int32)],
                        lerp(v, fx, fy, fz, f))
                return c2

            lax.fori_loop(0, VPC, comb, 0)

        pltpu.sync_copy(obuf, out_hbm.at[pl.ds(base, C)])
        return carry

    lax.fori_loop(0, NCH, chunk_body, 0)


@functools.cache
def _make_sc_forward():
    return functools.partial(
        pl.kernel,
        mesh=plsc.VectorSubcoreMesh(core_axis_name="c", subcore_axis_name="s"),
        out_type=jax.ShapeDtypeStruct((N, 8), jnp.float32),
        compiler_params=pltpu.CompilerParams(
            needs_layout_passes=False, use_tc_tiling_on_sc=False),
        scratch_types=[
            pltpu.VMEM((L0_ROWS, 8), jnp.float32),
            pltpu.VMEM((C * 3,), jnp.float32),
            pltpu.VMEM((C,), jnp.int32),
            pltpu.VMEM((3 * C,), jnp.float32),
            pltpu.VMEM((8 * C,), jnp.int32),
            pltpu.VMEM((4 * C,), jnp.float32),
            pltpu.VMEM((8 * C,), jnp.int32),
            pltpu.VMEM((4 * C,), jnp.float32),
            pltpu.VMEM((8 * C,), jnp.int32),
            pltpu.VMEM((4 * C,), jnp.float32),
            pltpu.VMEM((8 * C, 8), jnp.float32),
            pltpu.VMEM((8 * C, 8), jnp.float32),
            pltpu.VMEM((8 * C, 8), jnp.float32),
            pltpu.VMEM((C, 8), jnp.float32),
            pltpu.SemaphoreType.DMA,
            pltpu.SemaphoreType.DMA,
            pltpu.SemaphoreType.DMA,
        ],
    )(_sc_body)


def kernel(input, flattened_params):
    tab = flattened_params.reshape(N_V8, 8)
    return _make_sc_forward()(input.reshape(N * 3), tab)


# speedup vs baseline: 1.5877x; 1.0853x over previous
import functools

import jax
import jax.numpy as jnp
from jax import lax
from jax.experimental import pallas as pl
from jax.experimental.pallas import tpu as pltpu
from jax.experimental.pallas import tpu_sc as plsc

N = 1048576
N_V8 = 4792320
L0_ROWS = 8192
LOD0 = (32, 0)
LOD123 = ((64, 32768), (128, 294912), (256, 2392064))

NW = 32
PW = N // NW
C = 128
NCH = PW // C
VPC = C // 16


def _corner_offsets(R):
    return (0, 1, R, R + 1, R * R, R * R + 1, R * R + R, R * R + R + 1)


def _sc_body(x_hbm, tab_hbm, out_hbm, l0tab, *bufs):
    sets = (bufs[:16], bufs[16:])
    wid = lax.axis_index("s") * 2 + lax.axis_index("c")
    lanes = jnp.arange(16, dtype=jnp.int32)

    pltpu.sync_copy(tab_hbm.at[pl.ds(0, L0_ROWS)], l0tab)

    def gen_chunk(k, B):
        (xbuf, pb0, fr0, i1, fr1, i2, fr2, i3, fr3) = B[:9]
        base = wid * PW + k * C
        pltpu.sync_copy(x_hbm.at[pl.ds(base * 3, C * 3)], xbuf)

        def gen(i, c2):
            l0 = i * 16
            p = (lanes + l0) * 3
            x = plsc.load_gather(xbuf, [p])
            y = plsc.load_gather(xbuf, [p + 1])
            z = plsc.load_gather(xbuf, [p + 2])

            def prep(R, off):
                px = (x * 0.5 + 0.5) * (R - 1)
                py = (y * 0.5 + 0.5) * (R - 1)
                pz = (z * 0.5 + 0.5) * (R - 1)
                x0 = jnp.minimum(px.astype(jnp.int32), R - 2)
                y0 = jnp.minimum(py.astype(jnp.int32), R - 2)
                z0 = jnp.minimum(pz.astype(jnp.int32), R - 2)
                fx = px - x0.astype(jnp.float32)
                fy = py - y0.astype(jnp.float32)
                fz = pz - z0.astype(jnp.float32)
                b = (x0 * R + y0) * R + z0 + off
                return b, z0, fx, fy, fz

            R, off = LOD0
            b, _, fx, fy, fz = prep(R, off)
            fr0[pl.ds(l0, 16)] = fx
            fr0[pl.ds(C + l0, 16)] = fy
            fr0[pl.ds(2 * C + l0, 16)] = fz
            pb0[pl.ds(l0, 16)] = b

            for (R, off), ib, fb in zip(LOD123, (i1, i2, i3), (fr1, fr2, fr3)):
                b, z0, fx, fy, fz = prep(R, off)
                fb[pl.ds(l0, 16)] = fx
                fb[pl.ds(C + l0, 16)] = fy
                fb[pl.ds(2 * C + l0, 16)] = fz
                fb[pl.ds(3 * C + l0, 16)] = (
                    jnp.bitwise_and(z0, 3).astype(jnp.float32))
                for ci, d in enumerate(_corner_offsets(R)):
                    ib[pl.ds(ci * C + l0, 16)] = jnp.right_shift(b + d, 2)
            return c2

        lax.fori_loop(0, VPC, gen, 0)

    def fire(B):
        (i1, i2, i3) = (B[3], B[5], B[7])
        (r1, r2, r3, _, s1, s2, s3) = B[9:]
        for ib, rb, sm in ((i1, r1, s1), (i2, r2, s2), (i3, r3, s3)):
            pltpu.async_copy(tab_hbm.at[ib], rb, sm)

    def wait(B):
        (i1, i2, i3) = (B[3], B[5], B[7])
        (r1, r2, r3, _, s1, s2, s3) = B[9:]
        for ib, rb, sm in ((i1, r1, s1), (i2, r2, s2), (i3, r3, s3)):
            pltpu.make_async_copy(tab_hbm.at[ib], rb, sm).wait()

    def lerp(v, fx, fy, fz, f):
        c00 = v[0][f] + fz * (v[1][f] - v[0][f])
        c01 = v[2][f] + fz * (v[3][f] - v[2][f])
        c10 = v[4][f] + fz * (v[5][f] - v[4][f])
        c11 = v[6][f] + fz * (v[7][f] - v[6][f])
        c0 = c00 + fy * (c01 - c00)
        c1_ = c10 + fy * (c11 - c10)
        return c0 + fx * (c1_ - c0)

    def comb_chunk(k, B):
        (_, pb0, fr0, _, fr1, _, fr2, _, fr3) = B[:9]
        (r1, r2, r3, obuf) = B[9:13]
        base = wid * PW + k * C

        def comb0(i, c2):
            l0 = i * 16
            lane = lanes + l0
            fx = fr0[pl.ds(l0, 16)]
            fy = fr0[pl.ds(C + l0, 16)]
            fz = fr0[pl.ds(2 * C + l0, 16)]
            pb = pb0[pl.ds(l0, 16)]
            v = []
            for d in _corner_offsets(LOD0[0]):
                e = pb + d
                row = jnp.right_shift(e, 2)
                col = jnp.bitwise_and(e, 3) * 2
                v.append((plsc.load_gather(l0tab, [row, col]),
                          plsc.load_gather(l0tab, [row, col + 1])))
            for f in range(2):
                plsc.store_scatter(
                    obuf, [lane, jnp.full((16,), f, jnp.int32)],
                    lerp(v, fx, fy, fz, f))
            return c2

        lax.fori_loop(0, VPC, comb0, 0)

        for li, (fb, rb) in enumerate(((fr1, r1), (fr2, r2), (fr3, r3))):

            def comb(i, c2, li=li, fb=fb, rb=rb):
                l0 = i * 16
                lane = lanes + l0
                fx = fb[pl.ds(l0, 16)]
                fy = fb[pl.ds(C + l0, 16)]
                fz = fb[pl.ds(2 * C + l0, 16)]
                zlow = fb[pl.ds(3 * C + l0, 16)].astype(jnp.int32)
                czero = zlow * 2
                cone = jnp.bitwise_and(zlow + 1, 3) * 2
                v = []
                for c in range(8):
                    r = lane + c * C
                    col = cone if (c & 1) else czero
                    v.append((plsc.load_gather(rb, [r, col]),
                              plsc.load_gather(rb, [r, col + 1])))
                for f in range(2):
                    plsc.store_scatter(
                        obuf,
                        [lane, jnp.full((16,), 2 * (li + 1) + f, jnp.int32)],
                        lerp(v, fx, fy, fz, f))
                return c2

            lax.fori_loop(0, VPC, comb, 0)

        pltpu.sync_copy(obuf, out_hbm.at[pl.ds(base, C)])

    gen_chunk(0, sets[0])
    fire(sets[0])

    def pair_body(k2, carry):
        ka = 2 * k2
        gen_chunk(ka + 1, sets[1])
        fire(sets[1])
        wait(sets[0])
        comb_chunk(ka, sets[0])

        @pl.when(k2 < NCH // 2 - 1)
        def _():
            gen_chunk(ka + 2, sets[0])
            fire(sets[0])

        wait(sets[1])
        comb_chunk(ka + 1, sets[1])
        return carry

    lax.fori_loop(0, NCH // 2, pair_body, 0)


def _phase_scratch():
    return [
        pltpu.VMEM((C * 3,), jnp.float32),
        pltpu.VMEM((C,), jnp.int32),
        pltpu.VMEM((3 * C,), jnp.float32),
        pltpu.VMEM((8 * C,), jnp.int32),
        pltpu.VMEM((4 * C,), jnp.float32),
        pltpu.VMEM((8 * C,), jnp.int32),
        pltpu.VMEM((4 * C,), jnp.float32),
        pltpu.VMEM((8 * C,), jnp.int32),
        pltpu.VMEM((4 * C,), jnp.float32),
        pltpu.VMEM((8 * C, 8), jnp.float32),
        pltpu.VMEM((8 * C, 8), jnp.float32),
        pltpu.VMEM((8 * C, 8), jnp.float32),
        pltpu.VMEM((C, 8), jnp.float32),
        pltpu.SemaphoreType.DMA,
        pltpu.SemaphoreType.DMA,
        pltpu.SemaphoreType.DMA,
    ]


@functools.cache
def _make_sc_forward():
    return functools.partial(
        pl.kernel,
        mesh=plsc.VectorSubcoreMesh(core_axis_name="c", subcore_axis_name="s"),
        out_type=jax.ShapeDtypeStruct((N, 8), jnp.float32),
        compiler_params=pltpu.CompilerParams(
            needs_layout_passes=False, use_tc_tiling_on_sc=False),
        scratch_types=(
            [pltpu.VMEM((L0_ROWS, 8), jnp.float32)]
            + _phase_scratch() + _phase_scratch()),
    )(_sc_body)


def kernel(input, flattened_params):
    tab = flattened_params.reshape(N_V8, 8)
    return _make_sc_forward()(input.reshape(N * 3), tab)


# speedup vs baseline: 1.5892x; 1.0009x over previous
import functools

import jax
import jax.numpy as jnp
from jax import lax
from jax.experimental import pallas as pl
from jax.experimental.pallas import tpu as pltpu
from jax.experimental.pallas import tpu_sc as plsc

N = 1048576
N_V8 = 4792320
L0_ROWS = 8192
LOD0 = (32, 0)
LOD123 = ((64, 32768), (128, 294912), (256, 2392064))

NW = 32
PW = N // NW
C = 128
NCH = PW // C
VPC = C // 16


def _corner_offsets(R):
    return (0, 1, R, R + 1, R * R, R * R + 1, R * R + R, R * R + R + 1)


def _sc_body(x_hbm, tab_hbm, out_hbm, l0tab, *bufs):
    sets = (bufs[:10], bufs[10:])
    wid = lax.axis_index("s") * 2 + lax.axis_index("c")
    lanes = jnp.arange(16, dtype=jnp.int32)

    pltpu.sync_copy(tab_hbm.at[pl.ds(0, L0_ROWS)], l0tab)

    def gen_chunk(k, B):
        (xbuf, pb0, fr0, ib, fr1, fr2, fr3) = B[:7]
        base = wid * PW + k * C
        pltpu.sync_copy(x_hbm.at[pl.ds(base * 3, C * 3)], xbuf)

        def gen(i, c2):
            l0 = i * 16
            p = (lanes + l0) * 3
            x = plsc.load_gather(xbuf, [p])
            y = plsc.load_gather(xbuf, [p + 1])
            z = plsc.load_gather(xbuf, [p + 2])

            def prep(R, off):
                px = (x * 0.5 + 0.5) * (R - 1)
                py = (y * 0.5 + 0.5) * (R - 1)
                pz = (z * 0.5 + 0.5) * (R - 1)
                x0 = jnp.minimum(px.astype(jnp.int32), R - 2)
                y0 = jnp.minimum(py.astype(jnp.int32), R - 2)
                z0 = jnp.minimum(pz.astype(jnp.int32), R - 2)
                fx = px - x0.astype(jnp.float32)
                fy = py - y0.astype(jnp.float32)
                fz = pz - z0.astype(jnp.float32)
                b = (x0 * R + y0) * R + z0 + off
                return b, z0, fx, fy, fz

            R, off = LOD0
            b, _, fx, fy, fz = prep(R, off)
            fr0[pl.ds(l0, 16)] = fx
            fr0[pl.ds(C + l0, 16)] = fy
            fr0[pl.ds(2 * C + l0, 16)] = fz
            pb0[pl.ds(l0, 16)] = b

            for li, ((R, off), fb) in enumerate(
                    zip(LOD123, (fr1, fr2, fr3))):
                b, z0, fx, fy, fz = prep(R, off)
                fb[pl.ds(l0, 16)] = fx
                fb[pl.ds(C + l0, 16)] = fy
                fb[pl.ds(2 * C + l0, 16)] = fz
                fb[pl.ds(3 * C + l0, 16)] = (
                    jnp.bitwise_and(z0, 3).astype(jnp.float32))
                for ci, d in enumerate(_corner_offsets(R)):
                    ib[pl.ds((li * 8 + ci) * C + l0, 16)] = (
                        jnp.right_shift(b + d, 2))
            return c2

        lax.fori_loop(0, VPC, gen, 0)

    def fire(B):
        (ib, rb, sm) = (B[3], B[7], B[9])
        pltpu.async_copy(tab_hbm.at[ib], rb, sm)

    def wait(B):
        (ib, rb, sm) = (B[3], B[7], B[9])
        pltpu.make_async_copy(tab_hbm.at[ib], rb, sm).wait()

    def lerp(v, fx, fy, fz, f):
        c00 = v[0][f] + fz * (v[1][f] - v[0][f])
        c01 = v[2][f] + fz * (v[3][f] - v[2][f])
        c10 = v[4][f] + fz * (v[5][f] - v[4][f])
        c11 = v[6][f] + fz * (v[7][f] - v[6][f])
        c0 = c00 + fy * (c01 - c00)
        c1_ = c10 + fy * (c11 - c10)
        return c0 + fx * (c1_ - c0)

    def comb_chunk(k, B):
        (_, pb0, fr0, _, fr1, fr2, fr3, rb, obuf) = B[:9]
        base = wid * PW + k * C

        def comb0(i, c2):
            l0 = i * 16
            lane = lanes + l0
            fx = fr0[pl.ds(l0, 16)]
            fy = fr0[pl.ds(C + l0, 16)]
            fz = fr0[pl.ds(2 * C + l0, 16)]
            pb = pb0[pl.ds(l0, 16)]
            v = []
            for d in _corner_offsets(LOD0[0]):
                e = pb + d
                row = jnp.right_shift(e, 2)
                col = jnp.bitwise_and(e, 3) * 2
                v.append((plsc.load_gather(l0tab, [row, col]),
                          plsc.load_gather(l0tab, [row, col + 1])))
            for f in range(2):
                plsc.store_scatter(
                    obuf, [lane, jnp.full((16,), f, jnp.int32)],
                    lerp(v, fx, fy, fz, f))
            return c2

        lax.fori_loop(0, VPC, comb0, 0)

        for li, fb in enumerate((fr1, fr2, fr3)):

            def comb(i, c2, li=li, fb=fb):
                l0 = i * 16
                lane = lanes + l0
                fx = fb[pl.ds(l0, 16)]
                fy = fb[pl.ds(C + l0, 16)]
                fz = fb[pl.ds(2 * C + l0, 16)]
                zlow = fb[pl.ds(3 * C + l0, 16)].astype(jnp.int32)
                czero = zlow * 2
                cone = jnp.bitwise_and(zlow + 1, 3) * 2
                v = []
                for c in range(8):
                    r = lane + (li * 8 + c) * C
                    col = cone if (c & 1) else czero
                    v.append((plsc.load_gather(rb, [r, col]),
                              plsc.load_gather(rb, [r, col + 1])))
                for f in range(2):
                    plsc.store_scatter(
                        obuf,
                        [lane, jnp.full((16,), 2 * (li + 1) + f, jnp.int32)],
                        lerp(v, fx, fy, fz, f))
                return c2

            lax.fori_loop(0, VPC, comb, 0)

        pltpu.sync_copy(obuf, out_hbm.at[pl.ds(base, C)])

    gen_chunk(0, sets[0])
    fire(sets[0])

    def pair_body(k2, carry):
        ka = 2 * k2
        gen_chunk(ka + 1, sets[1])
        fire(sets[1])
        wait(sets[0])
        comb_chunk(ka, sets[0])

        @pl.when(k2 < NCH // 2 - 1)
        def _():
            gen_chunk(ka + 2, sets[0])
            fire(sets[0])

        wait(sets[1])
        comb_chunk(ka + 1, sets[1])
        return carry

    lax.fori_loop(0, NCH // 2, pair_body, 0)


def _phase_scratch():
    return [
        pltpu.VMEM((C * 3,), jnp.float32),
        pltpu.VMEM((C,), jnp.int32),
        pltpu.VMEM((3 * C,), jnp.float32),
        pltpu.VMEM((24 * C,), jnp.int32),
        pltpu.VMEM((4 * C,), jnp.float32),
        pltpu.VMEM((4 * C,), jnp.float32),
        pltpu.VMEM((4 * C,), jnp.float32),
        pltpu.VMEM((24 * C, 8), jnp.float32),
        pltpu.VMEM((C, 8), jnp.float32),
        pltpu.SemaphoreType.DMA,
    ]


@functools.cache
def _make_sc_forward():
    return functools.partial(
        pl.kernel,
        mesh=plsc.VectorSubcoreMesh(core_axis_name="c", subcore_axis_name="s"),
        out_type=jax.ShapeDtypeStruct((N, 8), jnp.float32),
        compiler_params=pltpu.CompilerParams(
            needs_layout_passes=False, use_tc_tiling_on_sc=False),
        scratch_types=(
            [pltpu.VMEM((L0_ROWS, 8), jnp.float32)]
            + _phase_scratch() + _phase_scratch()),
    )(_sc_body)


def kernel(input, flattened_params):
    tab = flattened_params.reshape(N_V8, 8)
    return _make_sc_forward()(input.reshape(N * 3), tab)
